# baseline (speedup 1.0000x reference)
import jax
import jax.numpy as jnp
from jax import lax
from jax.experimental import pallas as pl
from jax.experimental.pallas import tpu as pltpu

E = 16
D = 2048
N = 8192
TM = 512
GRID = N // TM
EP = 128


def _router_body(x_ref, w_ref, gates_ref, aux_ref, p_acc, f_acc):
    i = pl.program_id(0)

    @pl.when(i == 0)
    def _init():
        p_acc[...] = jnp.zeros_like(p_acc)
        f_acc[...] = jnp.zeros_like(f_acc)

    logits = jnp.dot(x_ref[...].astype(jnp.bfloat16),
                     w_ref[...].astype(jnp.bfloat16),
                     preferred_element_type=jnp.float32)
    ii = lax.broadcasted_iota(jnp.int32, (TM, EP), 1).astype(jnp.float32)
    neg = jnp.float32(-jnp.inf)
    big = jnp.float32(EP)
    lg = jnp.where(ii < E, logits, neg)
    m1 = jnp.max(lg, axis=1, keepdims=True)
    i1 = jnp.min(jnp.where(lg == m1, ii, big), axis=1, keepdims=True)
    lg2 = jnp.where(ii == i1, neg, lg)
    m2 = jnp.max(lg2, axis=1, keepdims=True)
    i2 = jnp.min(jnp.where(lg2 == m2, ii, big), axis=1, keepdims=True)
    e2 = jnp.exp(m2 - m1)
    g1 = 1.0 / (1.0 + e2)
    g2 = e2 * g1
    gates = jnp.where(ii == i1, g1, jnp.where(ii == i2, g2, 0.0))
    gates_ref[...] = gates[:, :E].T
    ex = jnp.exp(lg - m1)
    p = ex / jnp.sum(ex, axis=1, keepdims=True)
    fr = ((ii == i1).astype(jnp.float32)
          + ((ii == i2) & (g2 > 0)).astype(jnp.float32))
    p_acc[...] += jnp.sum(p, axis=0, keepdims=True)
    f_acc[...] += jnp.sum(fr, axis=0, keepdims=True)

    @pl.when(i == GRID - 1)
    def _fini():
        aux_ref[0, 0] = (E / (N * N)) * jnp.sum(p_acc[...] * f_acc[...])


def _run(x, w_pad, interpret=False):
    gates, aux = pl.pallas_call(
        _router_body,
        grid=(GRID,),
        in_specs=[pl.BlockSpec((TM, D), lambda i: (i, 0)),
                  pl.BlockSpec((D, EP), lambda i: (0, 0))],
        out_specs=[pl.BlockSpec((E, TM), lambda i: (0, i)),
                   pl.BlockSpec(memory_space=pltpu.SMEM)],
        out_shape=[jax.ShapeDtypeStruct((E, N), jnp.float32),
                   jax.ShapeDtypeStruct((1, 1), jnp.float32)],
        scratch_shapes=[pltpu.VMEM((1, EP), jnp.float32),
                        pltpu.VMEM((1, EP), jnp.float32)],
        interpret=interpret,
    )(x, w_pad)
    return gates.T, aux[0, 0]


def kernel(x, w_gate, w_noise):
    w_pad = jnp.pad(w_gate, ((0, 0), (0, EP - E)))
    return _run(x, w_pad)


# speedup vs baseline: 1.1573x; 1.1573x over previous
import jax
import jax.numpy as jnp
from jax import lax
from jax.experimental import pallas as pl
from jax.experimental.pallas import tpu as pltpu

E = 16
D = 2048
N = 8192
TM = 1024
GRID = N // TM
EP = 128


def _router_body(x_ref, w_ref, gates_ref, aux_ref, p_acc, f_acc):
    i = pl.program_id(0)

    @pl.when(i == 0)
    def _init():
        p_acc[...] = jnp.zeros_like(p_acc)
        f_acc[...] = jnp.zeros_like(f_acc)

    logits = jnp.dot(x_ref[...].astype(jnp.bfloat16),
                     w_ref[...].astype(jnp.bfloat16),
                     preferred_element_type=jnp.float32)
    ii = lax.broadcasted_iota(jnp.int32, (TM, EP), 1).astype(jnp.float32)
    neg = jnp.float32(-jnp.inf)
    big = jnp.float32(EP)
    lg = jnp.where(ii < E, logits, neg)
    m1 = jnp.max(lg, axis=1, keepdims=True)
    i1 = jnp.min(jnp.where(lg == m1, ii, big), axis=1, keepdims=True)
    lg2 = jnp.where(ii == i1, neg, lg)
    m2 = jnp.max(lg2, axis=1, keepdims=True)
    i2 = jnp.min(jnp.where(lg2 == m2, ii, big), axis=1, keepdims=True)
    e2 = jnp.exp(m2 - m1)
    g1 = 1.0 / (1.0 + e2)
    g2 = e2 * g1
    gates = jnp.where(ii == i1, g1, jnp.where(ii == i2, g2, 0.0))
    gates_ref[...] = gates[:, :E].T
    ex = jnp.exp(lg - m1)
    p = ex / jnp.sum(ex, axis=1, keepdims=True)
    fr = ((ii == i1).astype(jnp.float32)
          + ((ii == i2) & (g2 > 0)).astype(jnp.float32))
    p_acc[...] += jnp.sum(p, axis=0, keepdims=True)
    f_acc[...] += jnp.sum(fr, axis=0, keepdims=True)

    @pl.when(i == GRID - 1)
    def _fini():
        aux_ref[0, 0] = (E / (N * N)) * jnp.sum(p_acc[...] * f_acc[...])


def _run(x, w_pad, interpret=False):
    gates, aux = pl.pallas_call(
        _router_body,
        grid=(GRID,),
        in_specs=[pl.BlockSpec((TM, D), lambda i: (i, 0)),
                  pl.BlockSpec((D, EP), lambda i: (0, 0))],
        out_specs=[pl.BlockSpec((E, TM), lambda i: (0, i)),
                   pl.BlockSpec(memory_space=pltpu.SMEM)],
        out_shape=[jax.ShapeDtypeStruct((E, N), jnp.float32),
                   jax.ShapeDtypeStruct((1, 1), jnp.float32)],
        scratch_shapes=[pltpu.VMEM((1, EP), jnp.float32),
                        pltpu.VMEM((1, EP), jnp.float32)],
        interpret=interpret,
    )(x, w_pad)
    return gates.T, aux[0, 0]


def kernel(x, w_gate, w_noise):
    w_pad = jnp.pad(w_gate, ((0, 0), (0, EP - E)))
    return _run(x, w_pad)
